# baseline (speedup 1.0000x reference)
import functools

import jax
import jax.numpy as jnp
from jax import lax
from jax.experimental import pallas as pl
from jax.experimental.pallas import tpu as pltpu
from jax.experimental.pallas import tpu_sc as plsc

TOPK = 5
POOL = 30
PLEN = 8
DIM = 768
B = 1024
IHALF = PLEN // 2
HALF = IHALF * DIM

NC = 2
NS = 16
NW = NC * NS
BPW = B // NW
CHUNK = 8
NCH = BPW // CHUNK
NSLOT = 4


def _topk_body(xq_ref, ek_ref, idx_ref):
    ek = ek_ref[...]
    norm = jnp.sqrt(jnp.sum(ek * ek, axis=1, keepdims=True))
    ekn = ek / jnp.maximum(norm, 1e-12)
    q = xq_ref[...]
    qn = jnp.sqrt(jnp.sum(q * q, axis=1, keepdims=True))
    q = q / jnp.maximum(qn, 1e-12)
    s = lax.dot_general(
        q, ekn, (((1,), (1,)), ((), ())),
        preferred_element_type=jnp.float32)
    iota = lax.broadcasted_iota(jnp.int32, s.shape, 1)
    cols = []
    for _ in range(TOPK):
        m = jnp.max(s, axis=1, keepdims=True)
        it = jnp.min(jnp.where(s == m, iota, POOL), axis=1, keepdims=True)
        cols.append(it)
        s = jnp.where(iota == it, -jnp.inf, s)
    cols += [jnp.zeros((B, 1), jnp.int32)] * (8 - TOPK)
    idx_ref[...] = jnp.concatenate(cols, axis=1)


_topk = pl.pallas_call(
    _topk_body,
    out_shape=jax.ShapeDtypeStruct((B, 8), jnp.int32),
)


def _gather_body(ekt, idx_hbm, pk_out, idx_v,
                 buf0, buf1, buf2, buf3,
                 gsem0, gsem1, gsem2, gsem3,
                 wsem0, wsem1, wsem2, wsem3):
    wid = lax.axis_index("s") * NC + lax.axis_index("c")
    b0 = wid * BPW
    pltpu.sync_copy(idx_hbm.at[:, pl.ds(NCH * wid, NCH), :], idx_v)

    bufs = (buf0, buf1, buf2, buf3)
    gsems = (gsem0, gsem1, gsem2, gsem3)
    wsems = (wsem0, wsem1, wsem2, wsem3)
    NU = TOPK * NCH

    def start_gather(u, slot):
        t, c = divmod(u, NCH)
        return pltpu.async_copy(ekt.at[idx_v.at[t, c]], bufs[slot], gsems[slot])

    def start_writes(u, slot):
        t, c = divmod(u, NCH)
        dst_b = pl.ds(b0 + c * CHUNK, CHUNK)
        return [
            pltpu.async_copy(
                bufs[slot].at[:, pl.ds(i * DIM, DIM)],
                pk_out.at[IHALF * t + i, dst_b, :],
                wsems[slot])
            for i in range(IHALF)
        ]

    pend_w = [None] * NSLOT
    g = [None] * NSLOT
    depth = NSLOT - 2
    for u in range(min(depth, NU)):
        g[u % NSLOT] = start_gather(u, u % NSLOT)
    for u in range(NU):
        slot = u % NSLOT
        pre = u + depth
        if pre < NU:
            ps = pre % NSLOT
            if pend_w[ps] is not None:
                for d in pend_w[ps]:
                    d.wait()
                pend_w[ps] = None
            g[ps] = start_gather(pre, ps)
        g[slot].wait()
        pend_w[slot] = start_writes(u, slot)
    for slot in range(NSLOT):
        if pend_w[slot] is not None:
            for d in pend_w[slot]:
                d.wait()


@functools.lru_cache(maxsize=1)
def _gather_call():
    return pl.kernel(
        _gather_body,
        out_type=jax.ShapeDtypeStruct((TOPK * IHALF, B, DIM), jnp.float32),
        mesh=plsc.VectorSubcoreMesh(
            core_axis_name="c", subcore_axis_name="s",
            num_cores=NC, num_subcores=NS),
        scratch_types=(
            [pltpu.VMEM((TOPK, NCH, CHUNK), jnp.int32)]
            + [pltpu.VMEM((CHUNK, HALF), jnp.float32)] * NSLOT
            + [pltpu.SemaphoreType.DMA] * (2 * NSLOT)
        ),
    )


NBLK = 4
BS = B // NBLK


def _evwr_body(idx_ref, tbl_ref, out_ref):
    pool_iota = lax.broadcasted_iota(jnp.int32, (BS, POOL), 1)
    for t in range(TOPK):
        oh = (idx_ref[:, t:t + 1] == pool_iota).astype(jnp.float32)
        for i in range(IHALF):
            seg = lax.dot_general(
                oh, tbl_ref[:, i, :], (((1,), (0,)), ((), ())),
                preferred_element_type=jnp.float32,
                precision=lax.Precision.HIGHEST)
            out_ref[IHALF * t + i, :, :] = seg


_evwriter = pl.pallas_call(
    _evwr_body,
    grid=(NBLK,),
    in_specs=[
        pl.BlockSpec((BS, 8), lambda j: (j, 0)),
        pl.BlockSpec((POOL, IHALF, DIM), lambda j: (0, 0, 0)),
    ],
    out_specs=pl.BlockSpec((TOPK * IHALF, BS, DIM), lambda j: (0, j, 0)),
    out_shape=jax.ShapeDtypeStruct((TOPK * IHALF, B, DIM), jnp.float32),
)


def kernel(x_query, x, e_k, e_p, layer_id):
    idx8 = _topk(x_query, e_k)
    idx = idx8[:, :TOPK].T.reshape(TOPK, B // CHUNK, CHUNK)
    ekt = e_p[:, :IHALF, :].reshape(POOL, HALF)
    pk = _gather_call()(ekt, idx)
    pv = _evwriter(idx8, e_p[:, IHALF:, :])
    ek_o = jnp.transpose(pk, (1, 0, 2))
    ev_o = jnp.transpose(pv, (1, 0, 2))
    return (ek_o, ev_o, jnp.float32(0.0), x)


# speedup vs baseline: 1.0136x; 1.0136x over previous
import functools

import jax
import jax.numpy as jnp
from jax import lax
from jax.experimental import pallas as pl
from jax.experimental.pallas import tpu as pltpu
from jax.experimental.pallas import tpu_sc as plsc

TOPK = 5
POOL = 30
PLEN = 8
DIM = 768
B = 1024
IHALF = PLEN // 2
HALF = IHALF * DIM

NC = 2
NS = 16
NW = NC * NS
BPW = B // NW
CHUNK = 16
NCH = BPW // CHUNK
NSLOT = 2


def _topk_body(xq_ref, ek_ref, idx_ref):
    ek = ek_ref[...]
    norm = jnp.sqrt(jnp.sum(ek * ek, axis=1, keepdims=True))
    ekn = ek / jnp.maximum(norm, 1e-12)
    q = xq_ref[...]
    qn = jnp.sqrt(jnp.sum(q * q, axis=1, keepdims=True))
    q = q / jnp.maximum(qn, 1e-12)
    s = lax.dot_general(
        q, ekn, (((1,), (1,)), ((), ())),
        preferred_element_type=jnp.float32)
    iota = lax.broadcasted_iota(jnp.int32, s.shape, 1)
    cols = []
    for _ in range(TOPK):
        m = jnp.max(s, axis=1, keepdims=True)
        it = jnp.min(jnp.where(s == m, iota, POOL), axis=1, keepdims=True)
        cols.append(it)
        s = jnp.where(iota == it, -jnp.inf, s)
    cols += [jnp.zeros((B, 1), jnp.int32)] * (8 - TOPK)
    idx_ref[...] = jnp.concatenate(cols, axis=1)


_topk = pl.pallas_call(
    _topk_body,
    out_shape=jax.ShapeDtypeStruct((B, 8), jnp.int32),
)


def _gather_body(ekt, idx_hbm, pk_out, idx_v,
                 buf0, buf1, gsem0, gsem1, wsem0, wsem1):
    wid = lax.axis_index("s") * NC + lax.axis_index("c")
    b0 = wid * BPW
    pltpu.sync_copy(idx_hbm.at[:, pl.ds(NCH * wid, NCH), :], idx_v)

    bufs = (buf0, buf1)
    gsems = (gsem0, gsem1)
    wsems = (wsem0, wsem1)
    NU = TOPK * NCH

    def start_gather(u, slot):
        t, c = divmod(u, NCH)
        return pltpu.async_copy(ekt.at[idx_v.at[t, c]], bufs[slot], gsems[slot])

    def start_writes(u, slot):
        t, c = divmod(u, NCH)
        dst_b = pl.ds(b0 + c * CHUNK, CHUNK)
        return [
            pltpu.async_copy(
                bufs[slot].at[:, pl.ds(i * DIM, DIM)],
                pk_out.at[IHALF * t + i, dst_b, :],
                wsems[slot])
            for i in range(IHALF)
        ]

    pend_w = [None] * NSLOT
    g = [None] * NSLOT
    depth = NSLOT - 1
    for u in range(min(depth, NU)):
        g[u % NSLOT] = start_gather(u, u % NSLOT)
    for u in range(NU):
        slot = u % NSLOT
        pre = u + depth
        if pre < NU:
            ps = pre % NSLOT
            if pend_w[ps] is not None:
                for d in pend_w[ps]:
                    d.wait()
                pend_w[ps] = None
            g[ps] = start_gather(pre, ps)
        g[slot].wait()
        pend_w[slot] = start_writes(u, slot)
    for slot in range(NSLOT):
        if pend_w[slot] is not None:
            for d in pend_w[slot]:
                d.wait()


@functools.lru_cache(maxsize=1)
def _gather_call():
    return pl.kernel(
        _gather_body,
        out_type=jax.ShapeDtypeStruct((TOPK * IHALF, B, DIM), jnp.float32),
        mesh=plsc.VectorSubcoreMesh(
            core_axis_name="c", subcore_axis_name="s",
            num_cores=NC, num_subcores=NS),
        scratch_types=(
            [pltpu.VMEM((TOPK, NCH, CHUNK), jnp.int32)]
            + [pltpu.VMEM((CHUNK, HALF), jnp.float32)] * NSLOT
            + [pltpu.SemaphoreType.DMA] * (2 * NSLOT)
        ),
    )


NBLK = 4
BS = B // NBLK


def _evwr_body(idx_ref, tbl_ref, out_ref):
    pool_iota = lax.broadcasted_iota(jnp.int32, (BS, POOL), 1)
    for t in range(TOPK):
        oh = (idx_ref[:, t:t + 1] == pool_iota).astype(jnp.float32)
        for i in range(IHALF):
            seg = lax.dot_general(
                oh, tbl_ref[:, i, :], (((1,), (0,)), ((), ())),
                preferred_element_type=jnp.float32,
                precision=lax.Precision.HIGHEST)
            out_ref[IHALF * t + i, :, :] = seg


_evwriter = pl.pallas_call(
    _evwr_body,
    grid=(NBLK,),
    in_specs=[
        pl.BlockSpec((BS, 8), lambda j: (j, 0)),
        pl.BlockSpec((POOL, IHALF, DIM), lambda j: (0, 0, 0)),
    ],
    out_specs=pl.BlockSpec((TOPK * IHALF, BS, DIM), lambda j: (0, j, 0)),
    out_shape=jax.ShapeDtypeStruct((TOPK * IHALF, B, DIM), jnp.float32),
)


def kernel(x_query, x, e_k, e_p, layer_id):
    idx8 = _topk(x_query, e_k)
    idx = idx8[:, :TOPK].T.reshape(TOPK, B // CHUNK, CHUNK)
    ekt = e_p[:, :IHALF, :].reshape(POOL, HALF)
    pk = _gather_call()(ekt, idx)
    pv = _evwriter(idx8, e_p[:, IHALF:, :])
    ek_o = jnp.transpose(pk, (1, 0, 2))
    ev_o = jnp.transpose(pv, (1, 0, 2))
    return (ek_o, ev_o, jnp.float32(0.0), x)
